# baseline (speedup 1.0000x reference)
import functools

import jax
import jax.numpy as jnp
from jax import lax
from jax.experimental import pallas as pl
from jax.experimental.pallas import tpu as pltpu
from jax.experimental.pallas import tpu_sc as plsc

N = 10000
DIN = 128
DH = 128
DC = 64
E = 320000

NC = 2
NS = 16
NW = NC * NS

N_PAD = 10240
ZROWS = N_PAD // NS

E_ALL = E + N
CHUNK = 128
NCHUNK = 84
EPT = NCHUNK * CHUNK
E_PAD = EPT * NW

_MESH = plsc.VectorSubcoreMesh(core_axis_name="c", subcore_axis_name="s")


def _make_sc_scatter(D):
    @functools.partial(
        pl.kernel,
        out_type=jax.ShapeDtypeStruct((NC, N_PAD, D), jnp.float32),
        mesh=_MESH,
        compiler_params=pltpu.CompilerParams(use_tc_tiling_on_sc=False),
        scratch_types=[
            pltpu.VMEM((CHUNK,), jnp.int32),
            pltpu.VMEM((CHUNK,), jnp.int32),
            pltpu.VMEM((CHUNK,), jnp.int32),
            pltpu.VMEM((CHUNK,), jnp.int32),
            pltpu.VMEM((CHUNK,), jnp.int32),
            pltpu.VMEM((CHUNK,), jnp.int32),
            pltpu.VMEM((CHUNK, D), jnp.float32),
            pltpu.VMEM((CHUNK, D), jnp.float32),
            pltpu.VMEM_SHARED((N_PAD, D), jnp.float32),
            pltpu.SemaphoreType.DMA,
            pltpu.SemaphoreType.DMA,
            pltpu.SemaphoreType.DMA,
            pltpu.SemaphoreType.DMA,
            pltpu.SemaphoreType.DMA,
        ],
    )
    def sc_scatter(g_hbm, src_hbm, dst_hbm, zeros_hbm, out_hbm,
                   si0, si1, si2, di0, di1, di2, rows0, rows1, acc_sh,
                   is0, is1, is2, gs0, gs1):
        sbufs = (si0, si1, si2)
        dbufs = (di0, di1, di2)
        rbufs = (rows0, rows1)
        isems = (is0, is1, is2)
        gsems = (gs0, gs1)
        cid = lax.axis_index("c")
        sid = lax.axis_index("s")
        wid = sid * NC + cid
        base = wid * EPT

        pltpu.sync_copy(zeros_hbm, acc_sh.at[pl.ds(sid * ZROWS, ZROWS)])
        plsc.subcore_barrier()

        def body(j, carry):
            off = base + j * CHUNK
            pltpu.sync_copy(src_hbm.at[pl.ds(off, CHUNK)], sbufs[0])
            pltpu.sync_copy(dst_hbm.at[pl.ds(off, CHUNK)], dbufs[0])
            pltpu.sync_copy(g_hbm.at[sbufs[0]], rbufs[0])
            pltpu.sync_copy(rbufs[0], acc_sh.at[dbufs[0]], add=True)
            return carry

        lax.fori_loop(0, NCHUNK, body, 0)
        plsc.subcore_barrier()
        pltpu.sync_copy(acc_sh.at[pl.ds(sid * ZROWS, ZROWS)],
                        out_hbm.at[cid, pl.ds(sid * ZROWS, ZROWS)])

    return sc_scatter


_sc_scatter_h = _make_sc_scatter(DH)
_sc_scatter_c = _make_sc_scatter(DC)


_DK = 14


@functools.partial(
    pl.kernel,
    out_type=jax.ShapeDtypeStruct((NC, N_PAD), jnp.float32),
    mesh=_MESH,
    compiler_params=pltpu.CompilerParams(use_tc_tiling_on_sc=False),
    scratch_types=[
        pltpu.VMEM((NCHUNK, CHUNK), jnp.int32),
        pltpu.VMEM((CHUNK,), jnp.float32),
        pltpu.VMEM_SHARED((N_PAD,), jnp.float32),
        pltpu.SemaphoreType.DMA,
    ],
)
def _sc_deg(dst_hbm, ones_hbm, zeros_hbm, out_hbm, dsts_v, ones_v, acc_sh, sem):
    cid = lax.axis_index("c")
    sid = lax.axis_index("s")
    wid = sid * NC + cid
    pltpu.sync_copy(dst_hbm.at[wid], dsts_v)
    pltpu.sync_copy(ones_hbm, ones_v)
    pltpu.sync_copy(zeros_hbm, acc_sh.at[pl.ds(sid * ZROWS, ZROWS)])
    plsc.subcore_barrier()

    def body(gg, carry):
        for k in range(_DK):
            pltpu.async_copy(ones_v, acc_sh.at[dsts_v.at[gg * _DK + k]],
                             sem, add=True)
        for k in range(_DK):
            pltpu.make_async_copy(ones_v, acc_sh.at[pl.ds(0, CHUNK)],
                                  sem).wait()
        return carry

    lax.fori_loop(0, NCHUNK // _DK, body, 0)
    plsc.subcore_barrier()
    pltpu.sync_copy(acc_sh.at[pl.ds(sid * ZROWS, ZROWS)],
                    out_hbm.at[cid, pl.ds(sid * ZROWS, ZROWS)])


_R = 1280


def _tc_a_body(x_ref, w_ref, deg_ref, g_ref, dinv_ref):
    deg = deg_ref[:, 0:1] + deg_ref[:, 1:2]
    dinv = jnp.where(deg > 0, lax.rsqrt(deg), 0.0)
    h = jnp.dot(x_ref[...], w_ref[...], preferred_element_type=jnp.float32)
    g_ref[...] = h * dinv
    dinv_ref[...] = dinv


def _tc_a(x_pad, W1, deg_pair):
    return pl.pallas_call(
        _tc_a_body,
        grid=(N_PAD // _R,),
        in_specs=[
            pl.BlockSpec((_R, DIN), lambda i: (i, 0)),
            pl.BlockSpec((DIN, DH), lambda i: (0, 0)),
            pl.BlockSpec((_R, 2), lambda i: (i, 0)),
        ],
        out_specs=[
            pl.BlockSpec((_R, DH), lambda i: (i, 0)),
            pl.BlockSpec((_R, 1), lambda i: (i, 0)),
        ],
        out_shape=[
            jax.ShapeDtypeStruct((N_PAD, DH), jnp.float32),
            jax.ShapeDtypeStruct((N_PAD, 1), jnp.float32),
        ],
    )(x_pad, W1, deg_pair)


def _tc_b_body(s_ref, dinv_ref, b_ref, w_ref, g2_ref):
    dinv = dinv_ref[...]
    z = (s_ref[0] + s_ref[1]) * dinv + b_ref[...]
    h = jnp.maximum(z, 0.0)
    h2 = jnp.dot(h, w_ref[...], preferred_element_type=jnp.float32)
    g2_ref[...] = h2 * dinv


def _tc_b(s1, dinv, b1, W3):
    return pl.pallas_call(
        _tc_b_body,
        grid=(N_PAD // _R,),
        in_specs=[
            pl.BlockSpec((NC, _R, DH), lambda i: (0, i, 0)),
            pl.BlockSpec((_R, 1), lambda i: (i, 0)),
            pl.BlockSpec((1, DH), lambda i: (0, 0)),
            pl.BlockSpec((DH, DC), lambda i: (0, 0)),
        ],
        out_specs=pl.BlockSpec((_R, DC), lambda i: (i, 0)),
        out_shape=jax.ShapeDtypeStruct((N_PAD, DC), jnp.float32),
    )(s1, dinv, b1, W3)


def _tc_c_body(s_ref, dinv_ref, b_ref, out_ref):
    out_ref[...] = (s_ref[0] + s_ref[1]) * dinv_ref[...] + b_ref[...]


def _tc_c(s2, dinv, b3):
    return pl.pallas_call(
        _tc_c_body,
        grid=(N_PAD // _R,),
        in_specs=[
            pl.BlockSpec((NC, _R, DC), lambda i: (0, i, 0)),
            pl.BlockSpec((_R, 1), lambda i: (i, 0)),
            pl.BlockSpec((1, DC), lambda i: (0, 0)),
        ],
        out_specs=pl.BlockSpec((_R, DC), lambda i: (i, 0)),
        out_shape=jax.ShapeDtypeStruct((N_PAD, DC), jnp.float32),
    )(s2, dinv, b3)


def kernel(x, edge_index, W1, b1, W3, b3):
    src = edge_index[0].astype(jnp.int32)
    dst = edge_index[1].astype(jnp.int32)
    loop = jnp.arange(N, dtype=jnp.int32)
    pad_s = jnp.zeros((E_PAD - E_ALL,), jnp.int32)
    pad_d = jnp.full((E_PAD - E_ALL,), N, jnp.int32)
    src_all = jnp.concatenate([src, loop, pad_s])
    dst_all = jnp.concatenate([dst, loop, pad_d])
    dst_3d = dst_all.reshape(NW, NCHUNK, CHUNK)

    zeros_h = jnp.zeros((ZROWS, DH), jnp.float32)
    zeros_c = jnp.zeros((ZROWS, DC), jnp.float32)
    zeros_1 = jnp.zeros((ZROWS,), jnp.float32)
    ones_k = jnp.ones((CHUNK,), jnp.float32)

    deg2 = _sc_deg(dst_3d, ones_k, zeros_1)
    deg_pair = deg2.T

    x_pad = jnp.zeros((N_PAD, DIN), jnp.float32).at[:N].set(x)
    g1, dinv = _tc_a(x_pad, W1, deg_pair)
    s1 = _sc_scatter_h(g1, src_all, dst_all, zeros_h)
    g2 = _tc_b(s1, dinv, b1.reshape(1, DH), W3)
    s2 = _sc_scatter_c(g2, src_all, dst_all, zeros_c)
    out = _tc_c(s2, dinv, b3.reshape(1, DC))
    return out[:N]


# speedup vs baseline: 1.1900x; 1.1900x over previous
import functools

import jax
import jax.numpy as jnp
from jax import lax
from jax.experimental import pallas as pl
from jax.experimental.pallas import tpu as pltpu
from jax.experimental.pallas import tpu_sc as plsc

N = 10000
DIN = 128
DH = 128
DC = 64
E = 320000

NC = 2
NS = 16
NW = NC * NS

N_PAD = 10240
ZROWS = N_PAD // NS

E_ALL = E + N
CHUNK = 128
NCHUNK = 84
EPT = NCHUNK * CHUNK
E_PAD = EPT * NW

_MESH = plsc.VectorSubcoreMesh(core_axis_name="c", subcore_axis_name="s")


def _make_sc_scatter(D):
    @functools.partial(
        pl.kernel,
        out_type=jax.ShapeDtypeStruct((NC, N_PAD, D), jnp.float32),
        mesh=_MESH,
        compiler_params=pltpu.CompilerParams(use_tc_tiling_on_sc=False),
        scratch_types=[
            pltpu.VMEM((CHUNK,), jnp.int32),
            pltpu.VMEM((CHUNK,), jnp.int32),
            pltpu.VMEM((CHUNK,), jnp.int32),
            pltpu.VMEM((CHUNK,), jnp.int32),
            pltpu.VMEM((CHUNK,), jnp.int32),
            pltpu.VMEM((CHUNK,), jnp.int32),
            pltpu.VMEM((CHUNK, D), jnp.float32),
            pltpu.VMEM((CHUNK, D), jnp.float32),
            pltpu.VMEM_SHARED((N_PAD, D), jnp.float32),
            pltpu.SemaphoreType.DMA,
            pltpu.SemaphoreType.DMA,
            pltpu.SemaphoreType.DMA,
            pltpu.SemaphoreType.DMA,
            pltpu.SemaphoreType.DMA,
        ],
    )
    def sc_scatter(g_hbm, src_hbm, dst_hbm, zeros_hbm, out_hbm,
                   si0, si1, si2, di0, di1, di2, rows0, rows1, acc_sh,
                   is0, is1, is2, gs0, gs1):
        sbufs = (si0, si1, si2)
        dbufs = (di0, di1, di2)
        rbufs = (rows0, rows1)
        isems = (is0, is1, is2)
        gsems = (gs0, gs1)
        cid = lax.axis_index("c")
        sid = lax.axis_index("s")
        wid = sid * NC + cid
        base = wid * EPT

        def issue_idx(j, q):
            pltpu.async_copy(src_hbm.at[pl.ds(base + j * CHUNK, CHUNK)],
                             sbufs[q], isems[q])
            pltpu.async_copy(dst_hbm.at[pl.ds(base + j * CHUNK, CHUNK)],
                             dbufs[q], isems[q])

        def wait_idx(q):
            pltpu.make_async_copy(src_hbm.at[pl.ds(0, CHUNK)],
                                  sbufs[q], isems[q]).wait()
            pltpu.make_async_copy(src_hbm.at[pl.ds(0, CHUNK)],
                                  dbufs[q], isems[q]).wait()

        def issue_gather(q, p):
            pltpu.async_copy(g_hbm.at[sbufs[q]], rbufs[p], gsems[p])

        def wait_gather(p):
            pltpu.make_async_copy(g_hbm.at[pl.ds(0, CHUNK)],
                                  rbufs[p], gsems[p]).wait()

        issue_idx(0, 0)
        issue_idx(1, 1)
        wait_idx(0)
        issue_gather(0, 0)
        pltpu.sync_copy(zeros_hbm, acc_sh.at[pl.ds(sid * ZROWS, ZROWS)])
        plsc.subcore_barrier()

        def do_chunk(j, t):
            @pl.when(j + 2 < NCHUNK)
            def _():
                issue_idx(j + 2, (t + 2) % 3)

            @pl.when(j + 1 < NCHUNK)
            def _():
                wait_idx((t + 1) % 3)
                issue_gather((t + 1) % 3, (t + 1) % 2)

            wait_gather(t % 2)
            pltpu.sync_copy(rbufs[t % 2], acc_sh.at[dbufs[t % 3]], add=True)

        def body(ii, carry):
            for t in range(6):
                do_chunk(ii * 6 + t, t)
            return carry

        lax.fori_loop(0, NCHUNK // 6, body, 0)
        plsc.subcore_barrier()
        pltpu.sync_copy(acc_sh.at[pl.ds(sid * ZROWS, ZROWS)],
                        out_hbm.at[cid, pl.ds(sid * ZROWS, ZROWS)])

    return sc_scatter


_sc_scatter_h = _make_sc_scatter(DH)
_sc_scatter_c = _make_sc_scatter(DC)


_DK = 14


@functools.partial(
    pl.kernel,
    out_type=jax.ShapeDtypeStruct((NC, N_PAD), jnp.float32),
    mesh=_MESH,
    compiler_params=pltpu.CompilerParams(use_tc_tiling_on_sc=False),
    scratch_types=[
        pltpu.VMEM((NCHUNK, CHUNK), jnp.int32),
        pltpu.VMEM((CHUNK,), jnp.float32),
        pltpu.VMEM_SHARED((N_PAD,), jnp.float32),
        pltpu.SemaphoreType.DMA,
    ],
)
def _sc_deg(dst_hbm, ones_hbm, zeros_hbm, out_hbm, dsts_v, ones_v, acc_sh, sem):
    cid = lax.axis_index("c")
    sid = lax.axis_index("s")
    wid = sid * NC + cid
    pltpu.sync_copy(dst_hbm.at[wid], dsts_v)
    pltpu.sync_copy(ones_hbm, ones_v)
    pltpu.sync_copy(zeros_hbm, acc_sh.at[pl.ds(sid * ZROWS, ZROWS)])
    plsc.subcore_barrier()

    def body(gg, carry):
        for k in range(_DK):
            pltpu.async_copy(ones_v, acc_sh.at[dsts_v.at[gg * _DK + k]],
                             sem, add=True)
        for k in range(_DK):
            pltpu.make_async_copy(ones_v, acc_sh.at[pl.ds(0, CHUNK)],
                                  sem).wait()
        return carry

    lax.fori_loop(0, NCHUNK // _DK, body, 0)
    plsc.subcore_barrier()
    pltpu.sync_copy(acc_sh.at[pl.ds(sid * ZROWS, ZROWS)],
                    out_hbm.at[cid, pl.ds(sid * ZROWS, ZROWS)])


_R = 1280


def _tc_a_body(x_ref, w_ref, deg_ref, g_ref, dinv_ref):
    deg = deg_ref[:, 0:1] + deg_ref[:, 1:2]
    dinv = jnp.where(deg > 0, lax.rsqrt(deg), 0.0)
    h = jnp.dot(x_ref[...], w_ref[...], preferred_element_type=jnp.float32)
    g_ref[...] = h * dinv
    dinv_ref[...] = dinv


def _tc_a(x_pad, W1, deg_pair):
    return pl.pallas_call(
        _tc_a_body,
        grid=(N_PAD // _R,),
        in_specs=[
            pl.BlockSpec((_R, DIN), lambda i: (i, 0)),
            pl.BlockSpec((DIN, DH), lambda i: (0, 0)),
            pl.BlockSpec((_R, 2), lambda i: (i, 0)),
        ],
        out_specs=[
            pl.BlockSpec((_R, DH), lambda i: (i, 0)),
            pl.BlockSpec((_R, 1), lambda i: (i, 0)),
        ],
        out_shape=[
            jax.ShapeDtypeStruct((N_PAD, DH), jnp.float32),
            jax.ShapeDtypeStruct((N_PAD, 1), jnp.float32),
        ],
    )(x_pad, W1, deg_pair)


def _tc_b_body(s_ref, dinv_ref, b_ref, w_ref, g2_ref):
    dinv = dinv_ref[...]
    z = (s_ref[0] + s_ref[1]) * dinv + b_ref[...]
    h = jnp.maximum(z, 0.0)
    h2 = jnp.dot(h, w_ref[...], preferred_element_type=jnp.float32)
    g2_ref[...] = h2 * dinv


def _tc_b(s1, dinv, b1, W3):
    return pl.pallas_call(
        _tc_b_body,
        grid=(N_PAD // _R,),
        in_specs=[
            pl.BlockSpec((NC, _R, DH), lambda i: (0, i, 0)),
            pl.BlockSpec((_R, 1), lambda i: (i, 0)),
            pl.BlockSpec((1, DH), lambda i: (0, 0)),
            pl.BlockSpec((DH, DC), lambda i: (0, 0)),
        ],
        out_specs=pl.BlockSpec((_R, DC), lambda i: (i, 0)),
        out_shape=jax.ShapeDtypeStruct((N_PAD, DC), jnp.float32),
    )(s1, dinv, b1, W3)


def _tc_c_body(s_ref, dinv_ref, b_ref, out_ref):
    out_ref[...] = (s_ref[0] + s_ref[1]) * dinv_ref[...] + b_ref[...]


def _tc_c(s2, dinv, b3):
    return pl.pallas_call(
        _tc_c_body,
        grid=(N_PAD // _R,),
        in_specs=[
            pl.BlockSpec((NC, _R, DC), lambda i: (0, i, 0)),
            pl.BlockSpec((_R, 1), lambda i: (i, 0)),
            pl.BlockSpec((1, DC), lambda i: (0, 0)),
        ],
        out_specs=pl.BlockSpec((_R, DC), lambda i: (i, 0)),
        out_shape=jax.ShapeDtypeStruct((N_PAD, DC), jnp.float32),
    )(s2, dinv, b3)


def kernel(x, edge_index, W1, b1, W3, b3):
    src = edge_index[0].astype(jnp.int32)
    dst = edge_index[1].astype(jnp.int32)
    loop = jnp.arange(N, dtype=jnp.int32)
    pad_s = jnp.zeros((E_PAD - E_ALL,), jnp.int32)
    pad_d = N + jnp.arange(E_PAD - E_ALL, dtype=jnp.int32) % (N_PAD - N)
    src_all = jnp.concatenate([src, loop, pad_s])
    dst_all = jnp.concatenate([dst, loop, pad_d])
    dst_3d = dst_all.reshape(NW, NCHUNK, CHUNK)

    zeros_h = jnp.zeros((ZROWS, DH), jnp.float32)
    zeros_c = jnp.zeros((ZROWS, DC), jnp.float32)
    zeros_1 = jnp.zeros((ZROWS,), jnp.float32)
    ones_k = jnp.ones((CHUNK,), jnp.float32)

    deg2 = _sc_deg(dst_3d, ones_k, zeros_1)
    deg_pair = deg2.T

    x_pad = jnp.zeros((N_PAD, DIN), jnp.float32).at[:N].set(x)
    g1, dinv = _tc_a(x_pad, W1, deg_pair)
    s1 = _sc_scatter_h(g1, src_all, dst_all, zeros_h)
    g2 = _tc_b(s1, dinv, b1.reshape(1, DH), W3)
    s2 = _sc_scatter_c(g2, src_all, dst_all, zeros_c)
    out = _tc_c(s2, dinv, b3.reshape(1, DC))
    return out[:N]


# speedup vs baseline: 1.7516x; 1.4720x over previous
import functools

import jax
import jax.numpy as jnp
from jax import lax
from jax.experimental import pallas as pl
from jax.experimental.pallas import tpu as pltpu
from jax.experimental.pallas import tpu_sc as plsc

N = 10000
DIN = 128
DH = 128
DC = 64
E = 320000

NC = 2
NS = 16
NW = NC * NS

N_PAD = 10240
ZROWS = N_PAD // NS

E_ALL = E + N
CHUNK = 128
NCHUNK = -(-E_ALL // (NW * CHUNK))
EPT = NCHUNK * CHUNK
E_PAD = EPT * NW

_MESH = plsc.VectorSubcoreMesh(core_axis_name="c", subcore_axis_name="s")


def _make_sc_scatter(D):
    @functools.partial(
        pl.kernel,
        out_type=jax.ShapeDtypeStruct((NC, N_PAD, D), jnp.float32),
        mesh=_MESH,
        compiler_params=pltpu.CompilerParams(use_tc_tiling_on_sc=False),
        scratch_types=[
            pltpu.VMEM((CHUNK,), jnp.int32),
            pltpu.VMEM((CHUNK,), jnp.int32),
            pltpu.VMEM((CHUNK, D), jnp.float32),
            pltpu.VMEM_SHARED((N_PAD, D), jnp.float32),
        ],
    )
    def sc_scatter(g_hbm, src_hbm, dst_hbm, zeros_hbm, out_hbm,
                   src_v, dst_v, rows_v, acc_sh):
        cid = lax.axis_index("c")
        sid = lax.axis_index("s")
        wid = sid * NC + cid
        pltpu.sync_copy(zeros_hbm, acc_sh.at[pl.ds(sid * ZROWS, ZROWS)])
        plsc.subcore_barrier()

        def body(j, carry):
            off = wid * EPT + j * CHUNK
            pltpu.sync_copy(src_hbm.at[pl.ds(off, CHUNK)], src_v)
            pltpu.sync_copy(dst_hbm.at[pl.ds(off, CHUNK)], dst_v)
            pltpu.sync_copy(g_hbm.at[src_v], rows_v)
            pltpu.sync_copy(rows_v, acc_sh.at[dst_v], add=True)
            return carry

        lax.fori_loop(0, NCHUNK, body, 0)
        plsc.subcore_barrier()
        pltpu.sync_copy(acc_sh.at[pl.ds(sid * ZROWS, ZROWS)],
                        out_hbm.at[cid, pl.ds(sid * ZROWS, ZROWS)])

    return sc_scatter


_sc_scatter_h = _make_sc_scatter(DH)
_sc_scatter_c = _make_sc_scatter(DC)


@functools.partial(
    pl.kernel,
    out_type=jax.ShapeDtypeStruct((NC, N_PAD), jnp.float32),
    mesh=_MESH,
    scratch_types=[
        pltpu.VMEM((CHUNK,), jnp.int32),
        pltpu.VMEM((CHUNK,), jnp.float32),
        pltpu.VMEM_SHARED((N_PAD,), jnp.float32),
    ],
)
def _sc_deg(dst_hbm, ones_hbm, zeros_hbm, out_hbm, dst_v, ones_v, acc_sh):
    cid = lax.axis_index("c")
    sid = lax.axis_index("s")
    wid = sid * NC + cid
    pltpu.sync_copy(zeros_hbm, acc_sh.at[pl.ds(sid * ZROWS, ZROWS)])
    pltpu.sync_copy(ones_hbm, ones_v)
    plsc.subcore_barrier()

    def body(j, carry):
        off = wid * EPT + j * CHUNK
        pltpu.sync_copy(dst_hbm.at[pl.ds(off, CHUNK)], dst_v)
        pltpu.sync_copy(ones_v, acc_sh.at[dst_v], add=True)
        return carry

    lax.fori_loop(0, NCHUNK, body, 0)
    plsc.subcore_barrier()
    pltpu.sync_copy(acc_sh.at[pl.ds(sid * ZROWS, ZROWS)],
                    out_hbm.at[cid, pl.ds(sid * ZROWS, ZROWS)])


_R = 1280


def _tc_a_body(x_ref, w_ref, deg_ref, g_ref, dinv_ref):
    deg = deg_ref[:, 0:1] + deg_ref[:, 1:2]
    dinv = jnp.where(deg > 0, lax.rsqrt(deg), 0.0)
    h = jnp.dot(x_ref[...], w_ref[...], preferred_element_type=jnp.float32)
    g_ref[...] = h * dinv
    dinv_ref[...] = dinv


def _tc_a(x_pad, W1, deg_pair):
    return pl.pallas_call(
        _tc_a_body,
        grid=(N_PAD // _R,),
        in_specs=[
            pl.BlockSpec((_R, DIN), lambda i: (i, 0)),
            pl.BlockSpec((DIN, DH), lambda i: (0, 0)),
            pl.BlockSpec((_R, 2), lambda i: (i, 0)),
        ],
        out_specs=[
            pl.BlockSpec((_R, DH), lambda i: (i, 0)),
            pl.BlockSpec((_R, 1), lambda i: (i, 0)),
        ],
        out_shape=[
            jax.ShapeDtypeStruct((N_PAD, DH), jnp.float32),
            jax.ShapeDtypeStruct((N_PAD, 1), jnp.float32),
        ],
    )(x_pad, W1, deg_pair)


def _tc_b_body(s_ref, dinv_ref, b_ref, w_ref, g2_ref):
    dinv = dinv_ref[...]
    z = (s_ref[0] + s_ref[1]) * dinv + b_ref[...]
    h = jnp.maximum(z, 0.0)
    h2 = jnp.dot(h, w_ref[...], preferred_element_type=jnp.float32)
    g2_ref[...] = h2 * dinv


def _tc_b(s1, dinv, b1, W3):
    return pl.pallas_call(
        _tc_b_body,
        grid=(N_PAD // _R,),
        in_specs=[
            pl.BlockSpec((NC, _R, DH), lambda i: (0, i, 0)),
            pl.BlockSpec((_R, 1), lambda i: (i, 0)),
            pl.BlockSpec((1, DH), lambda i: (0, 0)),
            pl.BlockSpec((DH, DC), lambda i: (0, 0)),
        ],
        out_specs=pl.BlockSpec((_R, DC), lambda i: (i, 0)),
        out_shape=jax.ShapeDtypeStruct((N_PAD, DC), jnp.float32),
    )(s1, dinv, b1, W3)


def _tc_c_body(s_ref, dinv_ref, b_ref, out_ref):
    out_ref[...] = (s_ref[0] + s_ref[1]) * dinv_ref[...] + b_ref[...]


def _tc_c(s2, dinv, b3):
    return pl.pallas_call(
        _tc_c_body,
        grid=(N_PAD // _R,),
        in_specs=[
            pl.BlockSpec((NC, _R, DC), lambda i: (0, i, 0)),
            pl.BlockSpec((_R, 1), lambda i: (i, 0)),
            pl.BlockSpec((1, DC), lambda i: (0, 0)),
        ],
        out_specs=pl.BlockSpec((_R, DC), lambda i: (i, 0)),
        out_shape=jax.ShapeDtypeStruct((N_PAD, DC), jnp.float32),
    )(s2, dinv, b3)


def kernel(x, edge_index, W1, b1, W3, b3):
    src = edge_index[0].astype(jnp.int32)
    dst = edge_index[1].astype(jnp.int32)
    loop = jnp.arange(N, dtype=jnp.int32)
    pad_s = jnp.zeros((E_PAD - E_ALL,), jnp.int32)
    pad_d = jnp.full((E_PAD - E_ALL,), N, jnp.int32)
    src_all = jnp.concatenate([src, loop, pad_s])
    dst_all = jnp.concatenate([dst, loop, pad_d])

    zeros_h = jnp.zeros((ZROWS, DH), jnp.float32)
    zeros_c = jnp.zeros((ZROWS, DC), jnp.float32)
    zeros_1 = jnp.zeros((ZROWS,), jnp.float32)
    ones_k = jnp.ones((CHUNK,), jnp.float32)

    deg2 = _sc_deg(dst_all, ones_k, zeros_1)
    deg_pair = deg2.T

    x_pad = jnp.zeros((N_PAD, DIN), jnp.float32).at[:N].set(x)
    g1, dinv = _tc_a(x_pad, W1, deg_pair)
    s1 = _sc_scatter_h(g1, src_all, dst_all, zeros_h)
    g2 = _tc_b(s1, dinv, b1.reshape(1, DH), W3)
    s2 = _sc_scatter_c(g2, src_all, dst_all, zeros_c)
    out = _tc_c(s2, dinv, b3.reshape(1, DC))
    return out[:N]
